# baseline (speedup 1.0000x reference)
import functools

import jax
import jax.numpy as jnp
from jax import lax
from jax.experimental import pallas as pl
from jax.experimental.pallas import tpu as pltpu
from jax.experimental.pallas import tpu_sc as plsc

D_MODEL = 2048
B_TOTAL = 8192
NUM_CORES = 2
NUM_SUBCORES = 16
NW = NUM_CORES * NUM_SUBCORES
B_PER_W = B_TOTAL // NW
CHUNK = 8
NCHUNK = B_PER_W // CHUNK
NBUF = 7
AHEAD = NBUF - 1

_mesh = plsc.VectorSubcoreMesh(core_axis_name="c", subcore_axis_name="s")


@functools.partial(
    pl.kernel,
    mesh=_mesh,
    out_type=jax.ShapeDtypeStruct((B_TOTAL, D_MODEL), jnp.float32),
    scratch_types=[
        pltpu.VMEM((B_PER_W,), jnp.int32),
        pltpu.VMEM((NBUF, CHUNK, D_MODEL), jnp.float32),
        pltpu.SemaphoreType.DMA((NBUF,)),
        pltpu.SemaphoreType.DMA((NBUF,)),
    ],
)
def _embed_sc(ids_hbm, table_hbm, out_hbm, idx_v, bufs, gsem, wsem):
    wid = lax.axis_index("s") * NUM_CORES + lax.axis_index("c")
    base = wid * B_PER_W
    pltpu.sync_copy(ids_hbm.at[pl.ds(base, B_PER_W)], idx_v)

    def start_gather(c):
        return pltpu.async_copy(
            table_hbm.at[idx_v.at[pl.ds(c * CHUNK, CHUNK)]],
            bufs.at[c % NBUF],
            gsem.at[c % NBUF],
        )

    def start_write(c):
        return pltpu.async_copy(
            bufs.at[c % NBUF],
            out_hbm.at[pl.ds(base + c * CHUNK, CHUNK)],
            wsem.at[c % NBUF],
        )

    gathers = [start_gather(c) for c in range(AHEAD)]
    writes = [None] * NCHUNK
    last_waited_write = -1
    for c in range(NCHUNK):
        gathers[c].wait()
        writes[c] = start_write(c)
        nxt = c + AHEAD
        if nxt < NCHUNK:
            if nxt - NBUF >= 0:
                writes[nxt - NBUF].wait()
                last_waited_write = nxt - NBUF
            gathers.append(start_gather(nxt))
    for c in range(last_waited_write + 1, NCHUNK):
        writes[c].wait()


def kernel(input_ids, wte):
    ids = input_ids.reshape(-1)
    out = _embed_sc(ids, wte)
    return out.reshape(input_ids.shape + (wte.shape[1],))


# speedup vs baseline: 1.0015x; 1.0015x over previous
import functools

import jax
import jax.numpy as jnp
from jax import lax
from jax.experimental import pallas as pl
from jax.experimental.pallas import tpu as pltpu
from jax.experimental.pallas import tpu_sc as plsc

D_MODEL = 2048
B_TOTAL = 8192
NUM_CORES = 2
NUM_SUBCORES = 16
NW = NUM_CORES * NUM_SUBCORES
B_PER_W = B_TOTAL // NW
CHUNK = 8
NCHUNK = B_PER_W // CHUNK
NBUF = 5
AHEAD = NBUF - 1

_mesh = plsc.VectorSubcoreMesh(core_axis_name="c", subcore_axis_name="s")


@functools.partial(
    pl.kernel,
    mesh=_mesh,
    out_type=jax.ShapeDtypeStruct((B_TOTAL, D_MODEL), jnp.float32),
    scratch_types=[
        pltpu.VMEM((B_PER_W,), jnp.int32),
        pltpu.VMEM((NBUF, CHUNK, D_MODEL), jnp.float32),
        pltpu.SemaphoreType.DMA((NBUF,)),
        pltpu.SemaphoreType.DMA((NBUF,)),
    ],
)
def _embed_sc(ids_hbm, table_hbm, out_hbm, idx_v, bufs, gsem, wsem):
    wid = lax.axis_index("s") * NUM_CORES + lax.axis_index("c")
    base = wid * B_PER_W
    pltpu.sync_copy(ids_hbm.at[pl.ds(base, B_PER_W)], idx_v)

    def start_gather(c):
        return pltpu.async_copy(
            table_hbm.at[idx_v.at[pl.ds(c * CHUNK, CHUNK)]],
            bufs.at[c % NBUF],
            gsem.at[c % NBUF],
        )

    def start_write(c):
        return pltpu.async_copy(
            bufs.at[c % NBUF],
            out_hbm.at[pl.ds(base + c * CHUNK, CHUNK)],
            wsem.at[c % NBUF],
        )

    gathers = [start_gather(c) for c in range(AHEAD)]
    writes = [None] * NCHUNK
    last_waited_write = -1
    for c in range(NCHUNK):
        gathers[c].wait()
        writes[c] = start_write(c)
        nxt = c + AHEAD
        if nxt < NCHUNK:
            if nxt - NBUF >= 0:
                writes[nxt - NBUF].wait()
                last_waited_write = nxt - NBUF
            gathers.append(start_gather(nxt))
    for c in range(last_waited_write + 1, NCHUNK):
        writes[c].wait()


def kernel(input_ids, wte):
    ids = input_ids.reshape(-1)
    out = _embed_sc(ids, wte)
    return out.reshape(input_ids.shape + (wte.shape[1],))


# speedup vs baseline: 1.0055x; 1.0040x over previous
import functools

import jax
import jax.numpy as jnp
from jax import lax
from jax.experimental import pallas as pl
from jax.experimental.pallas import tpu as pltpu
from jax.experimental.pallas import tpu_sc as plsc

D_MODEL = 2048
B_TOTAL = 8192
NUM_CORES = 2
NUM_SUBCORES = 16
NW = NUM_CORES * NUM_SUBCORES
B_PER_W = B_TOTAL // NW
CHUNK = 8
NCHUNK = B_PER_W // CHUNK
NBUF = 6
AHEAD = 3

_mesh = plsc.VectorSubcoreMesh(core_axis_name="c", subcore_axis_name="s")


@functools.partial(
    pl.kernel,
    mesh=_mesh,
    out_type=jax.ShapeDtypeStruct((B_TOTAL, D_MODEL), jnp.float32),
    scratch_types=[
        pltpu.VMEM((B_PER_W,), jnp.int32),
        pltpu.VMEM((NBUF, CHUNK, D_MODEL), jnp.float32),
        pltpu.SemaphoreType.DMA((NBUF,)),
        pltpu.SemaphoreType.DMA((NBUF,)),
    ],
)
def _embed_sc(ids_hbm, table_hbm, out_hbm, idx_v, bufs, gsem, wsem):
    wid = lax.axis_index("s") * NUM_CORES + lax.axis_index("c")
    base = wid * B_PER_W
    pltpu.sync_copy(ids_hbm.at[pl.ds(base, B_PER_W)], idx_v)

    def start_gather(c):
        return pltpu.async_copy(
            table_hbm.at[idx_v.at[pl.ds(c * CHUNK, CHUNK)]],
            bufs.at[c % NBUF],
            gsem.at[c % NBUF],
        )

    def start_write(c):
        return pltpu.async_copy(
            bufs.at[c % NBUF],
            out_hbm.at[pl.ds(base + c * CHUNK, CHUNK)],
            wsem.at[c % NBUF],
        )

    gathers = [start_gather(c) for c in range(AHEAD)]
    writes = [None] * NCHUNK
    last_waited_write = -1
    for c in range(NCHUNK):
        gathers[c].wait()
        writes[c] = start_write(c)
        nxt = c + AHEAD
        if nxt < NCHUNK:
            if nxt - NBUF >= 0:
                writes[nxt - NBUF].wait()
                last_waited_write = nxt - NBUF
            gathers.append(start_gather(nxt))
    for c in range(last_waited_write + 1, NCHUNK):
        writes[c].wait()


def kernel(input_ids, wte):
    ids = input_ids.reshape(-1)
    out = _embed_sc(ids, wte)
    return out.reshape(input_ids.shape + (wte.shape[1],))


# speedup vs baseline: 1.0138x; 1.0083x over previous
import functools

import jax
import jax.numpy as jnp
from jax import lax
from jax.experimental import pallas as pl
from jax.experimental.pallas import tpu as pltpu
from jax.experimental.pallas import tpu_sc as plsc

D_MODEL = 2048
B_TOTAL = 8192
NUM_CORES = 2
NUM_SUBCORES = 16
NW = NUM_CORES * NUM_SUBCORES
B_PER_W = B_TOTAL // NW
CHUNK = 8
NCHUNK = B_PER_W // CHUNK
NBUF = 6
AHEAD = NBUF - 1

_mesh = plsc.VectorSubcoreMesh(core_axis_name="c", subcore_axis_name="s")


@functools.partial(
    pl.kernel,
    mesh=_mesh,
    out_type=jax.ShapeDtypeStruct((B_TOTAL, D_MODEL), jnp.float32),
    scratch_types=[
        pltpu.VMEM((B_PER_W,), jnp.int32),
        pltpu.VMEM((NBUF, CHUNK, D_MODEL), jnp.float32),
        pltpu.SemaphoreType.DMA((NBUF,)),
        pltpu.SemaphoreType.DMA((NBUF,)),
    ],
)
def _embed_sc(ids_hbm, table_hbm, out_hbm, idx_v, bufs, gsem, wsem):
    wid = lax.axis_index("s") * NUM_CORES + lax.axis_index("c")
    base = wid * B_PER_W
    pltpu.sync_copy(ids_hbm.at[pl.ds(base, B_PER_W)], idx_v)

    def start_gather(c):
        return pltpu.async_copy(
            table_hbm.at[idx_v.at[pl.ds(c * CHUNK, CHUNK)]],
            bufs.at[c % NBUF],
            gsem.at[c % NBUF],
        )

    def start_write(c):
        return pltpu.async_copy(
            bufs.at[c % NBUF],
            out_hbm.at[pl.ds(base + c * CHUNK, CHUNK)],
            wsem.at[c % NBUF],
        )

    gathers = [start_gather(c) for c in range(AHEAD)]
    writes = [None] * NCHUNK
    last_waited_write = -1
    for c in range(NCHUNK):
        gathers[c].wait()
        writes[c] = start_write(c)
        nxt = c + AHEAD
        if nxt < NCHUNK:
            if nxt - NBUF >= 0:
                writes[nxt - NBUF].wait()
                last_waited_write = nxt - NBUF
            gathers.append(start_gather(nxt))
    for c in range(last_waited_write + 1, NCHUNK):
        writes[c].wait()


def kernel(input_ids, wte):
    ids = input_ids.reshape(-1)
    out = _embed_sc(ids, wte)
    return out.reshape(input_ids.shape + (wte.shape[1],))
